# baseline (speedup 1.0000x reference)
import functools

import jax
import jax.numpy as jnp
from jax import lax
from jax.experimental import pallas as pl
from jax.experimental.pallas import tpu as pltpu


def _yolo_body(o_ref, t_ref, tT_ref, a_ref, out_ref, *, Bn, An, Cc, Hh, Ww,
               T):
    b = pl.program_id(0)
    HW = Hh * Ww
    f32 = jnp.float32

    tt = t_ref[0]
    b1 = tt[:, 1:2]
    cx = tt[:, 0:1] * Ww
    cy = b1 * Hh
    w = tt[:, 2:3] * Ww
    h = tt[:, 3:4] * Hh
    tcl = tt[:, 4:5].astype(jnp.int32).astype(f32)

    aw = [a_ref[i, 0] for i in range(An)]
    ah = [a_ref[i, 1] for i in range(An)]

    def best_anchor(wv, hv):
        def anchor_iou(i):
            inter = jnp.minimum(wv, aw[i]) * jnp.minimum(hv, ah[i])
            return inter / (wv * hv + aw[i] * ah[i] - inter)

        i0, i1, i2 = anchor_iou(0), anchor_iou(1), anchor_iou(2)
        bst = jnp.where(i1 > i0, 1, 0)
        return jnp.where(i2 > jnp.maximum(i0, i1), 2, bst)

    best = best_anchor(w, h)

    cif = jnp.floor(cx)
    cjf = jnp.floor(cy)
    cjci = cjf.astype(jnp.int32) * Ww + cif.astype(jnp.int32)
    aw_best = jnp.where(best == 0, aw[0], jnp.where(best == 1, aw[1], aw[2]))
    ah_best = jnp.where(best == 0, ah[0], jnp.where(best == 1, ah[1], ah[2]))
    tb0 = cx - cif
    tb1 = cy - cjf
    tb2 = jnp.log(w / aw_best)
    tb3 = jnp.log(h / ah_best)

    ttT = tT_ref[0]
    b1T = ttT[1:2, :]
    cxT = ttT[0:1, :] * Ww
    cyT = b1T * Hh
    wT = ttT[2:3, :] * Ww
    hT = ttT[3:4, :] * Hh
    bestT = best_anchor(wT, hT)
    cjciT = jnp.floor(cyT).astype(jnp.int32) * Ww + jnp.floor(cxT).astype(
        jnp.int32)

    r_i = lax.broadcasted_iota(jnp.int32, (T, T), 0)
    c_i = lax.broadcasted_iota(jnp.int32, (T, T), 1)
    iszero = jnp.where(b1 == 0.0, 1.0, 0.0)
    iszeroT = jnp.where(b1T == 0.0, 1.0, 0.0)
    zcnt = jnp.sum(jnp.where(c_i <= r_i, iszeroT, 0.0), axis=1,
                   keepdims=True)
    valid_b = zcnt == 0.0
    zcntT = jnp.sum(jnp.where(r_i <= c_i, iszero, 0.0), axis=0,
                    keepdims=True)
    validT_b = zcntT == 0.0

    later_same = ((c_i > r_i) & (cjciT == cjci) & (bestT == best)
                  & validT_b)
    dupcnt = jnp.sum(jnp.where(later_same, 1.0, 0.0), axis=1, keepdims=True)
    islive_b = valid_b & (dupcnt == 0.0)

    hw_iota = lax.broadcasted_iota(jnp.int32, (1, HW), 1)
    gx = (hw_iota % Ww).astype(f32)
    gy = (hw_iota // Ww).astype(f32)

    Sb = w * h
    box_l = cx - w * 0.5
    box_r = cx + w * 0.5
    box_t = cy - h * 0.5
    box_b = cy + h * 0.5

    loss_box = jnp.float32(0.0)
    loss_conf = jnp.float32(0.0)
    loss_cls = jnp.float32(0.0)

    for a in range(An):
        base = a * (Cc + 5)
        tx = o_ref[0, base + 0:base + 1, :]
        ty = o_ref[0, base + 1:base + 2, :]
        tw = o_ref[0, base + 2:base + 3, :]
        th = o_ref[0, base + 3:base + 4, :]
        tcf = o_ref[0, base + 4:base + 5, :]
        sx = jax.nn.sigmoid(tx)
        sy = jax.nn.sigmoid(ty)
        ew = jnp.exp(tw)
        eh = jnp.exp(th)
        pc = jax.nn.sigmoid(tcf)

        qa = (An * b + a) // Bn
        awq = jnp.where(qa == 0, aw[0], jnp.where(qa == 1, aw[1], aw[2]))
        ahq = jnp.where(qa == 0, ah[0], jnp.where(qa == 1, ah[1], ah[2]))

        px = sx + gx
        py = sy + gy
        pw = ew * awq
        ph = eh * ahq
        Sa = pw * ph
        pl_ = px - pw * 0.5
        pr_ = px + pw * 0.5
        pt_ = py - ph * 0.5
        pb_ = py + ph * 0.5

        x1 = jnp.maximum(pl_, box_l)
        x2 = jnp.minimum(pr_, box_r)
        y1 = jnp.maximum(pt_, box_t)
        y2 = jnp.minimum(pb_, box_b)
        inter = jnp.maximum(x2 - x1, 0.0) * jnp.maximum(y2 - y1, 0.0)
        ig_pred = (3.0 * inter > Sa + Sb) & valid_b
        ign = jnp.max(jnp.where(ig_pred, 1.0, 0.0), axis=0, keepdims=True)

        islive_a = jnp.where(islive_b & (best == a), 1.0, 0.0)
        live = jnp.where((cjci == hw_iota) & (best == a) & islive_b,
                         1.0, 0.0).astype(f32)

        notign = 1.0 - ign
        p6 = jnp.concatenate([sx, sy, ew, eh, pc, notign], axis=0)
        dn = (((1,), (1,)), ((), ()))
        g = lax.dot_general(live, p6, dn, preferred_element_type=f32)
        for k, tbk in enumerate((tb0, tb1, tb2, tb3)):
            d = g[:, k:k + 1] - tbk
            loss_box += jnp.sum(islive_a * d * d)

        gpc = g[:, 4:5]
        dconf = gpc - 1.0
        loss_conf += (jnp.sum(pc * pc * notign)
                      - jnp.sum(islive_a * gpc * gpc * g[:, 5:6])
                      + jnp.sum(islive_a * dconf * dconf))

        cls = o_ref[0, base + 5:base + 5 + Cc, :]
        rows = lax.dot_general(live, cls, dn, preferred_element_type=f32)
        mx = jnp.max(rows, axis=1, keepdims=True)
        ssum = jnp.sum(jnp.exp(rows - mx), axis=1, keepdims=True)
        lse = mx + jnp.log(ssum)
        c_iota = lax.broadcasted_iota(jnp.int32, (T, Cc), 1).astype(f32)
        picked = jnp.sum(rows * jnp.where(c_iota == tcl, 1.0, 0.0),
                         axis=1, keepdims=True)
        loss_cls += jnp.sum(islive_a * (picked - lse))

    total = loss_box * 0.5 + loss_conf - loss_cls

    @pl.when(b == 0)
    def _():
        out_ref[:, :] = jnp.zeros((1, 1), jnp.float32)

    out_ref[:, :] = out_ref[:, :] + total


def kernel(output, target, anchors):
    Bn, ch, Hh, Ww = output.shape
    An = anchors.shape[0]
    Cc = ch // An - 5
    T = target.shape[1] // 5
    HW = Hh * Ww

    o3 = output.reshape(Bn, ch, HW)
    t3 = target.reshape(Bn, T, 5)
    t3T = jnp.swapaxes(t3, 1, 2)

    body = functools.partial(_yolo_body, Bn=Bn, An=An, Cc=Cc, Hh=Hh, Ww=Ww,
                             T=T)
    res = pl.pallas_call(
        body,
        grid=(Bn,),
        in_specs=[
            pl.BlockSpec((1, ch, HW), lambda b: (b, 0, 0)),
            pl.BlockSpec((1, T, 5), lambda b: (b, 0, 0)),
            pl.BlockSpec((1, 5, T), lambda b: (b, 0, 0)),
            pl.BlockSpec((An, 2), lambda b: (0, 0)),
        ],
        out_specs=pl.BlockSpec((1, 1), lambda b: (0, 0)),
        out_shape=jax.ShapeDtypeStruct((1, 1), jnp.float32),
        compiler_params=pltpu.CompilerParams(
            dimension_semantics=("arbitrary",)),
    )(o3, t3, t3T, anchors)
    return res[0, 0]


# speedup vs baseline: 1.0329x; 1.0329x over previous
import functools

import jax
import jax.numpy as jnp
from jax import lax
from jax.experimental import pallas as pl
from jax.experimental.pallas import tpu as pltpu

_NF = 13


def _yolo_body(o_ref, t_ref, tT_ref, a_ref, out_ref, m_ref, *, Bn, An, Cc,
               Hh, Ww, T):
    b = pl.program_id(0)
    HW = Hh * Ww
    f32 = jnp.float32

    aw = [a_ref[i, 0] for i in range(An)]
    ah = [a_ref[i, 1] for i in range(An)]

    def best_anchor(wv, hv):
        def anchor_iou(i):
            inter = jnp.minimum(wv, aw[i]) * jnp.minimum(hv, ah[i])
            return inter / (wv * hv + aw[i] * ah[i] - inter)

        i0, i1, i2 = anchor_iou(0), anchor_iou(1), anchor_iou(2)
        bst = jnp.where(i1 > i0, 1, 0)
        return jnp.where(i2 > jnp.maximum(i0, i1), 2, bst)

    @pl.when(b == 0)
    def _match():
        tt = t_ref[...]
        b1 = tt[:, :, 1:2]
        cx = tt[:, :, 0:1] * Ww
        cy = b1 * Hh
        w = tt[:, :, 2:3] * Ww
        h = tt[:, :, 3:4] * Hh
        tcl = tt[:, :, 4:5].astype(jnp.int32).astype(f32)

        best = best_anchor(w, h)
        cif = jnp.floor(cx)
        cjf = jnp.floor(cy)
        cjci = cjf * Ww + cif
        aw_b = jnp.where(best == 0, aw[0],
                         jnp.where(best == 1, aw[1], aw[2]))
        ah_b = jnp.where(best == 0, ah[0],
                         jnp.where(best == 1, ah[1], ah[2]))
        tb0 = cx - cif
        tb1 = cy - cjf
        tb2 = jnp.log(w / aw_b)
        tb3 = jnp.log(h / ah_b)

        ttT = tT_ref[...]
        b1T = ttT[:, 1:2, :]
        wT = ttT[:, 2:3, :] * Ww
        hT = ttT[:, 3:4, :] * Hh
        bestT = best_anchor(wT, hT)
        cjciT = (jnp.floor(b1T * Hh) * Ww
                 + jnp.floor(ttT[:, 0:1, :] * Ww))

        r_i = lax.broadcasted_iota(jnp.int32, (Bn, T, T), 1)
        c_i = lax.broadcasted_iota(jnp.int32, (Bn, T, T), 2)
        iszero = jnp.where(b1 == 0.0, 1.0, 0.0)
        iszeroT = jnp.where(b1T == 0.0, 1.0, 0.0)
        zcnt = jnp.sum(jnp.where(c_i <= r_i, iszeroT, 0.0), axis=2,
                       keepdims=True)
        valid_b = zcnt == 0.0
        zcntT = jnp.sum(jnp.where(r_i <= c_i, iszero, 0.0), axis=1,
                        keepdims=True)
        validT_b = zcntT == 0.0

        later = ((c_i > r_i) & (cjciT == cjci) & (bestT == best) & validT_b)
        dupcnt = jnp.sum(jnp.where(later, 1.0, 0.0), axis=2, keepdims=True)
        islive = jnp.where(valid_b & (dupcnt == 0.0), 1.0, 0.0)

        rhs3 = jnp.where(valid_b, w * h * (1.0 / 3.0), jnp.float32(3e38))
        m_ref[...] = jnp.concatenate(
            [islive, best.astype(f32), cjci, tb0, tb1, tb2, tb3, tcl,
             cx - w * 0.5, cx + w * 0.5, cy - h * 0.5, cy + h * 0.5, rhs3],
            axis=2)

    mm = m_ref[pl.ds(b, 1)][0]
    islive_f = mm[:, 0:1]
    best = mm[:, 1:2].astype(jnp.int32)
    cjci = mm[:, 2:3].astype(jnp.int32)
    tbs = [mm[:, 3 + k:4 + k] for k in range(4)]
    tcl = mm[:, 7:8]
    box_l = mm[:, 8:9]
    box_r = mm[:, 9:10]
    box_t = mm[:, 10:11]
    box_b = mm[:, 11:12]
    rhs3 = mm[:, 12:13]
    islive_b = islive_f == 1.0

    hw_iota = lax.broadcasted_iota(jnp.int32, (1, HW), 1)
    gx = (hw_iota % Ww).astype(f32)
    gy = (hw_iota // Ww).astype(f32)

    loss_box = jnp.float32(0.0)
    loss_conf = jnp.float32(0.0)
    loss_cls = jnp.float32(0.0)

    for a in range(An):
        base = a * (Cc + 5)
        sx = jax.nn.sigmoid(o_ref[0, base + 0:base + 1, :])
        sy = jax.nn.sigmoid(o_ref[0, base + 1:base + 2, :])
        ew = jnp.exp(o_ref[0, base + 2:base + 3, :])
        eh = jnp.exp(o_ref[0, base + 3:base + 4, :])
        pc = jax.nn.sigmoid(o_ref[0, base + 4:base + 5, :])

        qa = (An * b + a) // Bn
        awq = jnp.where(qa == 0, aw[0], jnp.where(qa == 1, aw[1], aw[2]))
        ahq = jnp.where(qa == 0, ah[0], jnp.where(qa == 1, ah[1], ah[2]))

        px = sx + gx
        py = sy + gy
        pw = ew * awq
        ph = eh * ahq
        Sa3 = pw * ph * (1.0 / 3.0)
        pl_ = px - pw * 0.5
        pr_ = px + pw * 0.5
        pt_ = py - ph * 0.5
        pb_ = py + ph * 0.5

        x1 = jnp.maximum(pl_, box_l)
        x2 = jnp.minimum(pr_, box_r)
        y1 = jnp.maximum(pt_, box_t)
        y2 = jnp.minimum(pb_, box_b)
        inter = jnp.maximum(x2 - x1, 0.0) * jnp.maximum(y2 - y1, 0.0)
        ign_b = jnp.any(inter > Sa3 + rhs3, axis=0, keepdims=True)
        notign = jnp.where(ign_b, 0.0, 1.0)

        islive_a = jnp.where(islive_b & (best == a), 1.0, 0.0)
        live = jnp.where((cjci == hw_iota) & (best == a) & islive_b,
                         1.0, 0.0).astype(f32)

        p6 = jnp.concatenate([sx, sy, ew, eh, pc, notign], axis=0)
        dn = (((1,), (1,)), ((), ()))
        g = lax.dot_general(live, p6, dn, preferred_element_type=f32)
        for k in range(4):
            d = g[:, k:k + 1] - tbs[k]
            loss_box += jnp.sum(islive_a * d * d)

        gpc = g[:, 4:5]
        dconf = gpc - 1.0
        loss_conf += (jnp.sum(pc * pc * notign)
                      - jnp.sum(islive_a * gpc * gpc * g[:, 5:6])
                      + jnp.sum(islive_a * dconf * dconf))

        cls = o_ref[0, base + 5:base + 5 + Cc, :]
        rows = lax.dot_general(live, cls, dn, preferred_element_type=f32)
        mx = jnp.max(rows, axis=1, keepdims=True)
        ssum = jnp.sum(jnp.exp(rows - mx), axis=1, keepdims=True)
        lse = mx + jnp.log(ssum)
        c_iota = lax.broadcasted_iota(jnp.int32, (T, Cc), 1).astype(f32)
        picked = jnp.sum(rows * jnp.where(c_iota == tcl, 1.0, 0.0),
                         axis=1, keepdims=True)
        loss_cls += jnp.sum(islive_a * (picked - lse))

    total = loss_box * 0.5 + loss_conf - loss_cls

    @pl.when(b == 0)
    def _():
        out_ref[:, :] = jnp.zeros((1, 1), jnp.float32)

    out_ref[:, :] = out_ref[:, :] + total


def kernel(output, target, anchors):
    Bn, ch, Hh, Ww = output.shape
    An = anchors.shape[0]
    Cc = ch // An - 5
    T = target.shape[1] // 5
    HW = Hh * Ww

    o3 = output.reshape(Bn, ch, HW)
    t3 = target.reshape(Bn, T, 5)
    t3T = jnp.swapaxes(t3, 1, 2)

    body = functools.partial(_yolo_body, Bn=Bn, An=An, Cc=Cc, Hh=Hh, Ww=Ww,
                             T=T)
    res = pl.pallas_call(
        body,
        grid=(Bn,),
        in_specs=[
            pl.BlockSpec((1, ch, HW), lambda b: (b, 0, 0)),
            pl.BlockSpec((Bn, T, 5), lambda b: (0, 0, 0)),
            pl.BlockSpec((Bn, 5, T), lambda b: (0, 0, 0)),
            pl.BlockSpec((An, 2), lambda b: (0, 0)),
        ],
        out_specs=pl.BlockSpec((1, 1), lambda b: (0, 0)),
        out_shape=jax.ShapeDtypeStruct((1, 1), jnp.float32),
        scratch_shapes=[pltpu.VMEM((Bn, T, _NF), jnp.float32)],
        compiler_params=pltpu.CompilerParams(
            dimension_semantics=("arbitrary",)),
    )(o3, t3, t3T, anchors)
    return res[0, 0]
